# Initial kernel scaffold; baseline (speedup 1.0000x reference)
import jax
import jax.numpy as jnp
from jax.experimental import pallas as pl


def kernel(R, Z, idx_i, idx_j, emb, Wrbf, Wi, bi, Wj, bj, riW1, rib1, riW2, rib2, u, Wm, bm, raW1, rab1, raW2, rab2, roW1, rob1, roW2, rob2, Wout):
    raise NotImplementedError("write your pallas kernel here")


# speedup vs baseline: 2.6747x; 2.6747x over previous
import dataclasses
import functools
import math

import jax
import jax.numpy as jnp
from jax import lax
from jax.experimental import pallas as pl
from jax.experimental.pallas import tpu as pltpu
from jax.experimental.pallas import tpu_sc as plsc

N = 10000
E = 160000
F = 128
K = 64
B = 3
RI = 2
RA = 2
RO = 1
CUT = 10.0

NC = 2
NS = 16
NW = NC * NS
E_PAD = 163840
EPW = E_PAD // NW
CHUNK = 128
NCHUNK = EPW // CHUNK
EPB = E_PAD // 128
N_PAD = 10240
RPW = N_PAD // NW
RPS = N_PAD // NS
TN = 1024
TE = 1024

_LOG2 = math.log(2.0)
_C0 = math.exp(-CUT)
_CD = (1.0 - math.exp(-CUT)) / (K - 1)
_WIDTH = (0.5 * K / (1.0 - math.exp(-CUT))) ** 2

_MESH = plsc.VectorSubcoreMesh(core_axis_name="c", subcore_axis_name="s")

_SC_CP = pltpu.CompilerParams()
if "needs_layout_passes" in pltpu.CompilerParams.__dataclass_fields__:
    _SC_CP = dataclasses.replace(_SC_CP, needs_layout_passes=False)


def _ssp(v):
    return jnp.maximum(v, 0.0) + jnp.log(1.0 + jnp.exp(-jnp.abs(v))) - _LOG2


def _sc_prep_body(rx_hbm, ry_hbm, rz_hbm, z_hbm, ii_hbm, jj_hbm, emb_hbm,
                  d2_hbm, x0_hbm,
                  rx_v, ry_v, rz_v, ii_v, jj_v, d2_v, z_v, xrows_v, sem):
    c = lax.axis_index("c")
    s = lax.axis_index("s")
    wid = s * NC + c
    pltpu.sync_copy(rx_hbm, rx_v)
    pltpu.sync_copy(ry_hbm, ry_v)
    pltpu.sync_copy(rz_hbm, rz_v)
    ebase = wid * EPW
    pltpu.sync_copy(ii_hbm.at[pl.ds(ebase, EPW)], ii_v)
    pltpu.sync_copy(jj_hbm.at[pl.ds(ebase, EPW)], jj_v)

    @pl.loop(0, EPW // 16)
    def _(ci):
        o = ci * 16
        iv = ii_v[pl.ds(o, 16)]
        jv = jj_v[pl.ds(o, 16)]
        dx = plsc.load_gather(rx_v, [iv]) - plsc.load_gather(rx_v, [jv])
        dy = plsc.load_gather(ry_v, [iv]) - plsc.load_gather(ry_v, [jv])
        dz = plsc.load_gather(rz_v, [iv]) - plsc.load_gather(rz_v, [jv])
        d2_v[pl.ds(o, 16)] = dx * dx + dy * dy + dz * dz

    pltpu.sync_copy(d2_v, d2_hbm.at[pl.ds(ebase, EPW)])
    nbase = wid * RPW
    pltpu.sync_copy(z_hbm.at[pl.ds(nbase, RPW)], z_v)
    pltpu.async_copy(emb_hbm.at[z_v], xrows_v, sem).wait()
    pltpu.sync_copy(xrows_v, x0_hbm.at[pl.ds(nbase, RPW)])


def _sc_prep(rx, ry, rz, zp, ii, jj, emb):
    k = pl.kernel(
        _sc_prep_body,
        out_type=[jax.ShapeDtypeStruct((E_PAD,), jnp.float32),
                  jax.ShapeDtypeStruct((N_PAD, F), jnp.float32)],
        mesh=_MESH,
        scratch_types=[pltpu.VMEM((N_PAD,), jnp.float32),
                       pltpu.VMEM((N_PAD,), jnp.float32),
                       pltpu.VMEM((N_PAD,), jnp.float32),
                       pltpu.VMEM((EPW,), jnp.int32),
                       pltpu.VMEM((EPW,), jnp.int32),
                       pltpu.VMEM((EPW,), jnp.float32),
                       pltpu.VMEM((RPW,), jnp.int32),
                       pltpu.VMEM((RPW, F), jnp.float32),
                       pltpu.SemaphoreType.DMA],
        compiler_params=_SC_CP,
    )
    return k(rx, ry, rz, zp, ii, jj, emb)


def _sc_gather_body(hj_hbm, jj_hbm, hjg_hbm, ib_v, rb_v, sem):
    c = lax.axis_index("c")
    s = lax.axis_index("s")
    wid = s * NC + c

    @pl.loop(0, NCHUNK)
    def _(kk):
        base = wid * EPW + kk * CHUNK
        pltpu.sync_copy(jj_hbm.at[pl.ds(base, CHUNK)], ib_v)
        pltpu.async_copy(hj_hbm.at[ib_v], rb_v, sem).wait()
        pltpu.sync_copy(rb_v, hjg_hbm.at[pl.ds(base, CHUNK)])


def _sc_gather(hj, jj):
    k = pl.kernel(
        _sc_gather_body,
        out_type=jax.ShapeDtypeStruct((E_PAD, F), jnp.float32),
        mesh=_MESH,
        scratch_types=[pltpu.VMEM((CHUNK,), jnp.int32),
                       pltpu.VMEM((CHUNK, F), jnp.float32),
                       pltpu.SemaphoreType.DMA],
    )
    return k(hj, jj)


def _sc_scatter_body(t_hbm, ii_hbm, xjp_hbm, acc_sh, zb_v, tb_v, ib_v):
    c = lax.axis_index("c")
    s = lax.axis_index("s")
    wid = s * NC + c

    @pl.loop(0, 128)
    def _(r):
        @pl.loop(0, 8)
        def _(q):
            zb_v[r, pl.ds(q * 16, 16)] = jnp.zeros((16,), jnp.float32)

    @pl.loop(0, RPS // 128)
    def _(kk):
        pltpu.sync_copy(zb_v, acc_sh.at[pl.ds(s * RPS + kk * 128, 128)])
    plsc.subcore_barrier()

    @pl.loop(0, NCHUNK)
    def _(kk):
        base = wid * EPW + kk * CHUNK
        pltpu.sync_copy(ii_hbm.at[pl.ds(base, CHUNK)], ib_v)
        pltpu.sync_copy(t_hbm.at[pl.ds(base, CHUNK)], tb_v)
        pltpu.sync_copy(tb_v, acc_sh.at[ib_v], add=True)
    plsc.subcore_barrier()

    pltpu.sync_copy(acc_sh.at[pl.ds(s * RPS, RPS)],
                    xjp_hbm.at[c, pl.ds(s * RPS, RPS)])


def _sc_scatter(t, ii):
    k = pl.kernel(
        _sc_scatter_body,
        out_type=jax.ShapeDtypeStruct((NC, N_PAD, F), jnp.float32),
        mesh=_MESH,
        scratch_types=[pltpu.VMEM_SHARED((N_PAD, F), jnp.float32),
                       pltpu.VMEM((128, F), jnp.float32),
                       pltpu.VMEM((CHUNK, F), jnp.float32),
                       pltpu.VMEM((CHUNK,), jnp.int32)],
    )
    return k(t, ii)


def _tc_t_body(d2_ref, hjg_ref, w_ref, t_ref):
    pid = pl.program_id(0)
    d2 = d2_ref[...]
    d = jnp.sqrt(d2 + 1e-12)
    r = d / CUT
    r2 = r * r
    r3 = r2 * r
    poly = 1.0 - 6.0 * r3 * r2 + 15.0 * r2 * r2 - 10.0 * r3
    eid = (pid * TE
           + lax.broadcasted_iota(jnp.int32, (8, 128), 0) * 128
           + lax.broadcasted_iota(jnp.int32, (8, 128), 1))
    cut = jnp.where((d < CUT) & (eid < E), poly, 0.0)
    en = jnp.exp(-d)
    ck = _C0 + _CD * lax.broadcasted_iota(jnp.int32, (K, 1), 0).astype(jnp.float32)
    w = w_ref[...]
    for rr in range(8):
        en_r = jnp.broadcast_to(en[rr:rr + 1, :], (K, 128))
        cut_r = jnp.broadcast_to(cut[rr:rr + 1, :], (K, 128))
        diff = en_r - ck
        rbf_t = cut_r * jnp.exp(-_WIDTH * diff * diff)
        g = lax.dot_general(rbf_t, w, (((0,), (0,)), ((), ())),
                            preferred_element_type=jnp.float32)
        sl = pl.ds(rr * 128, 128)
        t_ref[sl, :] = g * hjg_ref[sl, :]


def _tc_t(d2r, hjg, wrbf):
    return pl.pallas_call(
        _tc_t_body,
        grid=(E_PAD // TE,),
        in_specs=[pl.BlockSpec((8, 128), lambda i: (i, 0)),
                  pl.BlockSpec((TE, F), lambda i: (i, 0)),
                  pl.BlockSpec((K, F), lambda i: (0, 0))],
        out_specs=pl.BlockSpec((TE, F), lambda i: (i, 0)),
        out_shape=jax.ShapeDtypeStruct((E_PAD, F), jnp.float32),
    )(d2r, hjg, wrbf)


def _tc_d1_body(x_ref, wi_ref, bi_ref, wj_ref, bj_ref, xi_ref, hj_ref):
    xa = _ssp(x_ref[...])
    xi_ref[...] = _ssp(jnp.dot(xa, wi_ref[...],
                               preferred_element_type=jnp.float32) + bi_ref[...])
    hj_ref[...] = _ssp(jnp.dot(xa, wj_ref[...],
                               preferred_element_type=jnp.float32) + bj_ref[...])


def _tc_d1(x, wi, bi, wj, bj):
    return pl.pallas_call(
        _tc_d1_body,
        grid=(N_PAD // TN,),
        in_specs=[pl.BlockSpec((TN, F), lambda i: (i, 0)),
                  pl.BlockSpec((F, F), lambda i: (0, 0)),
                  pl.BlockSpec((1, F), lambda i: (0, 0)),
                  pl.BlockSpec((F, F), lambda i: (0, 0)),
                  pl.BlockSpec((1, F), lambda i: (0, 0))],
        out_specs=[pl.BlockSpec((TN, F), lambda i: (i, 0)),
                   pl.BlockSpec((TN, F), lambda i: (i, 0))],
        out_shape=[jax.ShapeDtypeStruct((N_PAD, F), jnp.float32),
                   jax.ShapeDtypeStruct((N_PAD, F), jnp.float32)],
    )(x, wi, bi, wj, bj)


def _mm(a, w_ref):
    return jnp.dot(a, w_ref[...], preferred_element_type=jnp.float32)


def _tc_d2_body(has_prev, xi_ref, xj0_ref, xj1_ref, x_ref, eq_ref,
                ri10_ref, ri11_ref, ri20_ref, ri21_ref,
                rib10_ref, rib11_ref, rib20_ref, rib21_ref,
                u_ref, wm_ref, bm_ref,
                ra10_ref, ra11_ref, ra20_ref, ra21_ref,
                rab10_ref, rab11_ref, rab20_ref, rab21_ref,
                ro1_ref, ro2_ref, rob1_ref, rob2_ref,
                wout_ref, *rest):
    if has_prev:
        last_ref = rest[0]
        xo_ref, eqo_ref, o2_ref, nh_ref = rest[1:]
    else:
        xo_ref, eqo_ref, o2_ref = rest

    m = xi_ref[...] + xj0_ref[...] + xj1_ref[...]
    for w1, b1, w2, b2 in ((ri10_ref, rib10_ref, ri20_ref, rib20_ref),
                           (ri11_ref, rib11_ref, ri21_ref, rib21_ref)):
        ma = _ssp(m)
        m = m + _mm(_ssp(_mm(ma, w1) + b1[...]), w2) + b2[...]
    m = _ssp(m)
    x = u_ref[...] * x_ref[...] + _mm(m, wm_ref) + bm_ref[...]
    for w1, b1, w2, b2 in ((ra10_ref, rab10_ref, ra20_ref, rab20_ref),
                           (ra11_ref, rab11_ref, ra21_ref, rab21_ref)):
        xa2 = _ssp(x)
        x = x + _mm(_ssp(_mm(xa2, w1) + b1[...]), w2) + b2[...]
    xo_ref[...] = x
    o = x + _mm(_ssp(_mm(_ssp(x), ro1_ref) + rob1_ref[...]), ro2_ref) + rob2_ref[...]
    out = _mm(_ssp(o), wout_ref)
    eqo_ref[...] = eq_ref[...] + out
    o2 = out * out
    o2_ref[...] = o2
    if has_prev:
        pid = pl.program_id(0)
        rows = pid * TN + lax.broadcasted_iota(jnp.int32, (TN, 2), 0)
        ratio = jnp.where(rows < N, o2 / (o2 + last_ref[...] + 1e-7), 0.0)
        part = (jnp.sum(ratio) / (N * 2.0)).reshape(1, 1)

        @pl.when(pid == 0)
        def _():
            nh_ref[...] = part

        @pl.when(pid > 0)
        def _():
            nh_ref[...] += part


def _tc_d2(xi, xjp, x, eq, wts, last=None):
    has_prev = last is not None
    full = lambda shape: pl.BlockSpec(shape, lambda i: tuple(0 for _ in shape))
    in_specs = [pl.BlockSpec((TN, F), lambda i: (i, 0)),
                pl.BlockSpec((TN, F), lambda i: (i, 0)),
                pl.BlockSpec((TN, F), lambda i: (i, 0)),
                pl.BlockSpec((TN, F), lambda i: (i, 0)),
                pl.BlockSpec((TN, 2), lambda i: (i, 0))]
    wspecs = []
    for warr in wts:
        wspecs.append(full(tuple(warr.shape)))
    in_specs += wspecs
    out_specs = [pl.BlockSpec((TN, F), lambda i: (i, 0)),
                 pl.BlockSpec((TN, 2), lambda i: (i, 0)),
                 pl.BlockSpec((TN, 2), lambda i: (i, 0))]
    out_shape = [jax.ShapeDtypeStruct((N_PAD, F), jnp.float32),
                 jax.ShapeDtypeStruct((N_PAD, 2), jnp.float32),
                 jax.ShapeDtypeStruct((N_PAD, 2), jnp.float32)]
    args = [xi, xjp[0], xjp[1], x, eq] + list(wts)
    if has_prev:
        in_specs.append(pl.BlockSpec((TN, 2), lambda i: (i, 0)))
        out_specs.append(pl.BlockSpec((1, 1), lambda i: (0, 0)))
        out_shape.append(jax.ShapeDtypeStruct((1, 1), jnp.float32))
        args.append(last)
    return pl.pallas_call(
        functools.partial(_tc_d2_body, has_prev),
        grid=(N_PAD // TN,),
        in_specs=in_specs,
        out_specs=out_specs,
        out_shape=out_shape,
    )(*args)


def kernel(R, Z, idx_i, idx_j, emb, Wrbf, Wi, bi, Wj, bj, riW1, rib1, riW2,
           rib2, u, Wm, bm, raW1, rab1, raW2, rab2, roW1, rob1, roW2, rob2,
           Wout):
    f32 = jnp.float32
    pad_ids = (jnp.arange(E_PAD - E, dtype=jnp.int32) % N)
    ii = jnp.concatenate([idx_i.astype(jnp.int32), pad_ids])
    jj = jnp.concatenate([idx_j.astype(jnp.int32), pad_ids])
    zp = jnp.pad(Z.astype(jnp.int32), (0, N_PAD - N))
    rx = jnp.pad(R[:, 0], (0, N_PAD - N))
    ry = jnp.pad(R[:, 1], (0, N_PAD - N))
    rz = jnp.pad(R[:, 2], (0, N_PAD - N))

    d2, x0 = _sc_prep(rx, ry, rz, zp, ii, jj, emb)
    d2r = d2.reshape(EPB, 128)

    x = x0
    eq = jnp.zeros((N_PAD, 2), f32)
    nh = f32(0.0)
    last = None
    for b in range(B):
        xi, hj = _tc_d1(x, Wi[b], bi[b].reshape(1, F), Wj[b], bj[b].reshape(1, F))
        hjg = _sc_gather(hj, jj)
        t = _tc_t(d2r, hjg, Wrbf[b])
        xjp = _sc_scatter(t, ii)
        wts = (riW1[b, 0], riW1[b, 1], riW2[b, 0], riW2[b, 1],
               rib1[b, 0].reshape(1, F), rib1[b, 1].reshape(1, F),
               rib2[b, 0].reshape(1, F), rib2[b, 1].reshape(1, F),
               u[b].reshape(1, F), Wm[b], bm[b].reshape(1, F),
               raW1[b, 0], raW1[b, 1], raW2[b, 0], raW2[b, 1],
               rab1[b, 0].reshape(1, F), rab1[b, 1].reshape(1, F),
               rab2[b, 0].reshape(1, F), rab2[b, 1].reshape(1, F),
               roW1[b, 0], roW2[b, 0],
               rob1[b, 0].reshape(1, F), rob2[b, 0].reshape(1, F),
               Wout[b])
        res = _tc_d2(xi, xjp, x, eq, wts, last=last)
        if b == 0:
            x, eq, out2 = res
        else:
            x, eq, out2, nhp = res
            nh = nh + nhp[0, 0]
        last = out2
    return eq[:N, 0], eq[:N, 1], nh


# speedup vs baseline: 5.0903x; 1.9031x over previous
import dataclasses
import functools
import math

import jax
import jax.numpy as jnp
from jax import lax
from jax.experimental import pallas as pl
from jax.experimental.pallas import tpu as pltpu
from jax.experimental.pallas import tpu_sc as plsc

N = 10000
E = 160000
F = 128
K = 64
B = 3
RI = 2
RA = 2
RO = 1
CUT = 10.0

NC = 2
NS = 16
NW = NC * NS
E_PAD = 163840
EPW = E_PAD // NW
CHUNK = 64
NCHUNK = EPW // CHUNK
EPB = E_PAD // 128
N_PAD = 10240
RPW = N_PAD // NW
RPS = N_PAD // NS
TN = 1024
TE = 1024

_LOG2 = math.log(2.0)
_C0 = math.exp(-CUT)
_CD = (1.0 - math.exp(-CUT)) / (K - 1)
_WIDTH = (0.5 * K / (1.0 - math.exp(-CUT))) ** 2

_MESH = plsc.VectorSubcoreMesh(core_axis_name="c", subcore_axis_name="s")

_SC_CP = pltpu.CompilerParams()
if "needs_layout_passes" in pltpu.CompilerParams.__dataclass_fields__:
    _SC_CP = dataclasses.replace(_SC_CP, needs_layout_passes=False)


def _ssp(v):
    return jnp.maximum(v, 0.0) + jnp.log(1.0 + jnp.exp(-jnp.abs(v))) - _LOG2


def _sc_prep_body(rx_hbm, ry_hbm, rz_hbm, z_hbm, ii_hbm, jj_hbm, emb_hbm,
                  d2_hbm, x0_hbm,
                  rx_v, ry_v, rz_v, ii_v, jj_v, d2_v, z_v, xrows_v, sem):
    c = lax.axis_index("c")
    s = lax.axis_index("s")
    wid = s * NC + c
    pltpu.sync_copy(rx_hbm, rx_v)
    pltpu.sync_copy(ry_hbm, ry_v)
    pltpu.sync_copy(rz_hbm, rz_v)
    ebase = wid * EPW
    pltpu.sync_copy(ii_hbm.at[pl.ds(ebase, EPW)], ii_v)
    pltpu.sync_copy(jj_hbm.at[pl.ds(ebase, EPW)], jj_v)

    @pl.loop(0, EPW // 16)
    def _(ci):
        o = ci * 16
        iv = ii_v[pl.ds(o, 16)]
        jv = jj_v[pl.ds(o, 16)]
        dx = plsc.load_gather(rx_v, [iv]) - plsc.load_gather(rx_v, [jv])
        dy = plsc.load_gather(ry_v, [iv]) - plsc.load_gather(ry_v, [jv])
        dz = plsc.load_gather(rz_v, [iv]) - plsc.load_gather(rz_v, [jv])
        d2_v[pl.ds(o, 16)] = dx * dx + dy * dy + dz * dz

    pltpu.sync_copy(d2_v, d2_hbm.at[pl.ds(ebase, EPW)])
    nbase = wid * RPW
    pltpu.sync_copy(z_hbm.at[pl.ds(nbase, RPW)], z_v)
    pltpu.async_copy(emb_hbm.at[z_v], xrows_v, sem).wait()
    pltpu.sync_copy(xrows_v, x0_hbm.at[pl.ds(nbase, RPW)])


def _sc_prep(rx, ry, rz, zp, ii, jj, emb):
    k = pl.kernel(
        _sc_prep_body,
        out_type=[jax.ShapeDtypeStruct((E_PAD,), jnp.float32),
                  jax.ShapeDtypeStruct((N_PAD, F), jnp.float32)],
        mesh=_MESH,
        scratch_types=[pltpu.VMEM((N_PAD,), jnp.float32),
                       pltpu.VMEM((N_PAD,), jnp.float32),
                       pltpu.VMEM((N_PAD,), jnp.float32),
                       pltpu.VMEM((EPW,), jnp.int32),
                       pltpu.VMEM((EPW,), jnp.int32),
                       pltpu.VMEM((EPW,), jnp.float32),
                       pltpu.VMEM((RPW,), jnp.int32),
                       pltpu.VMEM((RPW, F), jnp.float32),
                       pltpu.SemaphoreType.DMA],
        compiler_params=_SC_CP,
    )
    return k(rx, ry, rz, zp, ii, jj, emb)


def _sc_agg_body(g_hbm, hj_hbm, iir_hbm, jjr_hbm, xjp_hbm,
                 acc_sh, jj_v, iib0, iib1, gb0, gb1, hb0, hb1,
                 gsem0, gsem1, hsem0, hsem1, isem0, isem1):
    c = lax.axis_index("c")
    s = lax.axis_index("s")
    wid = s * NC + c
    gb = (gb0, gb1)
    hb = (hb0, hb1)
    iib = (iib0, iib1)
    gsem = (gsem0, gsem1)
    hsem = (hsem0, hsem1)
    isem = (isem0, isem1)

    @pl.loop(0, CHUNK)
    def _(r):
        for q in range(8):
            gb0[r, pl.ds(q * 16, 16)] = jnp.zeros((16,), jnp.float32)

    @pl.loop(0, RPS // CHUNK)
    def _(kk):
        pltpu.sync_copy(gb0, acc_sh.at[pl.ds(s * RPS + kk * CHUNK, CHUNK)])

    pltpu.sync_copy(jjr_hbm.at[pl.ds(wid * (NCHUNK // 2), NCHUNK // 2)], jj_v)
    for b in range(2):
        base = wid * EPW + b * CHUNK
        pltpu.async_copy(g_hbm.at[pl.ds(base, CHUNK)], gb[b], gsem[b])
        pltpu.async_copy(hj_hbm.at[jj_v.at[0, pl.ds(b * CHUNK, CHUNK)]],
                         hb[b], hsem[b])
        pltpu.async_copy(iir_hbm.at[pl.ds(wid * NCHUNK + b, 1)], iib[b],
                         isem[b])
    plsc.subcore_barrier()

    @pl.loop(0, NCHUNK // 2)
    def _(rr):
        for b in range(2):
            k = rr * 2 + b
            pltpu.make_async_copy(g_hbm.at[pl.ds(0, CHUNK)], gb[b],
                                  gsem[b]).wait()
            pltpu.make_async_copy(g_hbm.at[pl.ds(0, CHUNK)], hb[b],
                                  hsem[b]).wait()

            @pl.loop(0, CHUNK)
            def _(r):
                for q in range(8):
                    sl = pl.ds(q * 16, 16)
                    gb[b][r, sl] = gb[b][r, sl] * hb[b][r, sl]

            pltpu.make_async_copy(iir_hbm.at[pl.ds(0, 1)], iib[b],
                                  isem[b]).wait()
            pltpu.sync_copy(gb[b], acc_sh.at[iib[b].at[0]], add=True)

            @pl.when(rr + 1 < NCHUNK // 2)
            def _():
                base2 = wid * EPW + (k + 2) * CHUNK
                pltpu.async_copy(g_hbm.at[pl.ds(base2, CHUNK)], gb[b], gsem[b])
                pltpu.async_copy(
                    hj_hbm.at[jj_v.at[rr + 1, pl.ds(b * CHUNK, CHUNK)]],
                    hb[b], hsem[b])
                pltpu.async_copy(iir_hbm.at[pl.ds(wid * NCHUNK + k + 2, 1)],
                                 iib[b], isem[b])

    plsc.subcore_barrier()
    pltpu.sync_copy(acc_sh.at[pl.ds(s * RPS, RPS)],
                    xjp_hbm.at[c, pl.ds(s * RPS, RPS)])


def _sc_agg(g, hj, iir, jjr):
    k = pl.kernel(
        _sc_agg_body,
        out_type=jax.ShapeDtypeStruct((NC, N_PAD, F), jnp.float32),
        mesh=_MESH,
        scratch_types=[pltpu.VMEM_SHARED((N_PAD, F), jnp.float32),
                       pltpu.VMEM((NCHUNK // 2, 128), jnp.int32),
                       pltpu.VMEM((1, CHUNK), jnp.int32),
                       pltpu.VMEM((1, CHUNK), jnp.int32),
                       pltpu.VMEM((CHUNK, F), jnp.float32),
                       pltpu.VMEM((CHUNK, F), jnp.float32),
                       pltpu.VMEM((CHUNK, F), jnp.float32),
                       pltpu.VMEM((CHUNK, F), jnp.float32),
                       pltpu.SemaphoreType.DMA,
                       pltpu.SemaphoreType.DMA,
                       pltpu.SemaphoreType.DMA,
                       pltpu.SemaphoreType.DMA,
                       pltpu.SemaphoreType.DMA,
                       pltpu.SemaphoreType.DMA],
        compiler_params=_SC_CP,
    )
    return k(g, hj, iir, jjr)


def _tc_g_body(d2_ref, w_ref, t_ref):
    pid = pl.program_id(0)
    d2 = d2_ref[...]
    d = jnp.sqrt(d2 + 1e-12)
    r = d / CUT
    r2 = r * r
    r3 = r2 * r
    poly = 1.0 - 6.0 * r3 * r2 + 15.0 * r2 * r2 - 10.0 * r3
    eid = (pid * TE
           + lax.broadcasted_iota(jnp.int32, (8, 128), 0) * 128
           + lax.broadcasted_iota(jnp.int32, (8, 128), 1))
    cut = jnp.where((d < CUT) & (eid < E), poly, 0.0)
    en = jnp.exp(-d)
    ck = _C0 + _CD * lax.broadcasted_iota(jnp.int32, (K, 1), 0).astype(jnp.float32)
    w = w_ref[...]
    for rr in range(8):
        en_r = jnp.broadcast_to(en[rr:rr + 1, :], (K, 128))
        cut_r = jnp.broadcast_to(cut[rr:rr + 1, :], (K, 128))
        diff = en_r - ck
        rbf_t = cut_r * jnp.exp(-_WIDTH * diff * diff)
        g = lax.dot_general(rbf_t, w, (((0,), (0,)), ((), ())),
                            preferred_element_type=jnp.float32)
        t_ref[pl.ds(rr * 128, 128), :] = g


def _tc_g(d2r, wrbf):
    return pl.pallas_call(
        _tc_g_body,
        grid=(E_PAD // TE,),
        in_specs=[pl.BlockSpec((8, 128), lambda i: (i, 0)),
                  pl.BlockSpec((K, F), lambda i: (0, 0))],
        out_specs=pl.BlockSpec((TE, F), lambda i: (i, 0)),
        out_shape=jax.ShapeDtypeStruct((E_PAD, F), jnp.float32),
    )(d2r, wrbf)


def _tc_d1_body(x_ref, wi_ref, bi_ref, wj_ref, bj_ref, xi_ref, hj_ref):
    xa = _ssp(x_ref[...])
    xi_ref[...] = _ssp(jnp.dot(xa, wi_ref[...],
                               preferred_element_type=jnp.float32) + bi_ref[...])
    hj_ref[...] = _ssp(jnp.dot(xa, wj_ref[...],
                               preferred_element_type=jnp.float32) + bj_ref[...])


def _tc_d1(x, wi, bi, wj, bj):
    return pl.pallas_call(
        _tc_d1_body,
        grid=(N_PAD // TN,),
        in_specs=[pl.BlockSpec((TN, F), lambda i: (i, 0)),
                  pl.BlockSpec((F, F), lambda i: (0, 0)),
                  pl.BlockSpec((1, F), lambda i: (0, 0)),
                  pl.BlockSpec((F, F), lambda i: (0, 0)),
                  pl.BlockSpec((1, F), lambda i: (0, 0))],
        out_specs=[pl.BlockSpec((TN, F), lambda i: (i, 0)),
                   pl.BlockSpec((TN, F), lambda i: (i, 0))],
        out_shape=[jax.ShapeDtypeStruct((N_PAD, F), jnp.float32),
                   jax.ShapeDtypeStruct((N_PAD, F), jnp.float32)],
    )(x, wi, bi, wj, bj)


def _mm(a, w_ref):
    return jnp.dot(a, w_ref[...], preferred_element_type=jnp.float32)


def _tc_d2_body(has_prev, xi_ref, xj0_ref, xj1_ref, x_ref, eq_ref,
                ri10_ref, ri11_ref, ri20_ref, ri21_ref,
                rib10_ref, rib11_ref, rib20_ref, rib21_ref,
                u_ref, wm_ref, bm_ref,
                ra10_ref, ra11_ref, ra20_ref, ra21_ref,
                rab10_ref, rab11_ref, rab20_ref, rab21_ref,
                ro1_ref, ro2_ref, rob1_ref, rob2_ref,
                wout_ref, *rest):
    if has_prev:
        last_ref = rest[0]
        xo_ref, eqo_ref, o2_ref, nh_ref = rest[1:]
    else:
        xo_ref, eqo_ref, o2_ref = rest

    m = xi_ref[...] + xj0_ref[...] + xj1_ref[...]
    for w1, b1, w2, b2 in ((ri10_ref, rib10_ref, ri20_ref, rib20_ref),
                           (ri11_ref, rib11_ref, ri21_ref, rib21_ref)):
        ma = _ssp(m)
        m = m + _mm(_ssp(_mm(ma, w1) + b1[...]), w2) + b2[...]
    m = _ssp(m)
    x = u_ref[...] * x_ref[...] + _mm(m, wm_ref) + bm_ref[...]
    for w1, b1, w2, b2 in ((ra10_ref, rab10_ref, ra20_ref, rab20_ref),
                           (ra11_ref, rab11_ref, ra21_ref, rab21_ref)):
        xa2 = _ssp(x)
        x = x + _mm(_ssp(_mm(xa2, w1) + b1[...]), w2) + b2[...]
    xo_ref[...] = x
    o = x + _mm(_ssp(_mm(_ssp(x), ro1_ref) + rob1_ref[...]), ro2_ref) + rob2_ref[...]
    out = _mm(_ssp(o), wout_ref)
    eqo_ref[...] = eq_ref[...] + out
    o2 = out * out
    o2_ref[...] = o2
    if has_prev:
        pid = pl.program_id(0)
        rows = pid * TN + lax.broadcasted_iota(jnp.int32, (TN, 2), 0)
        ratio = jnp.where(rows < N, o2 / (o2 + last_ref[...] + 1e-7), 0.0)
        part = (jnp.sum(ratio) / (N * 2.0)).reshape(1, 1)

        @pl.when(pid == 0)
        def _():
            nh_ref[...] = part

        @pl.when(pid > 0)
        def _():
            nh_ref[...] += part


def _tc_d2(xi, xjp, x, eq, wts, last=None):
    has_prev = last is not None
    full = lambda shape: pl.BlockSpec(shape, lambda i: tuple(0 for _ in shape))
    in_specs = [pl.BlockSpec((TN, F), lambda i: (i, 0)),
                pl.BlockSpec((TN, F), lambda i: (i, 0)),
                pl.BlockSpec((TN, F), lambda i: (i, 0)),
                pl.BlockSpec((TN, F), lambda i: (i, 0)),
                pl.BlockSpec((TN, 2), lambda i: (i, 0))]
    wspecs = []
    for warr in wts:
        wspecs.append(full(tuple(warr.shape)))
    in_specs += wspecs
    out_specs = [pl.BlockSpec((TN, F), lambda i: (i, 0)),
                 pl.BlockSpec((TN, 2), lambda i: (i, 0)),
                 pl.BlockSpec((TN, 2), lambda i: (i, 0))]
    out_shape = [jax.ShapeDtypeStruct((N_PAD, F), jnp.float32),
                 jax.ShapeDtypeStruct((N_PAD, 2), jnp.float32),
                 jax.ShapeDtypeStruct((N_PAD, 2), jnp.float32)]
    args = [xi, xjp[0], xjp[1], x, eq] + list(wts)
    if has_prev:
        in_specs.append(pl.BlockSpec((TN, 2), lambda i: (i, 0)))
        out_specs.append(pl.BlockSpec((1, 1), lambda i: (0, 0)))
        out_shape.append(jax.ShapeDtypeStruct((1, 1), jnp.float32))
        args.append(last)
    return pl.pallas_call(
        functools.partial(_tc_d2_body, has_prev),
        grid=(N_PAD // TN,),
        in_specs=in_specs,
        out_specs=out_specs,
        out_shape=out_shape,
    )(*args)


def kernel(R, Z, idx_i, idx_j, emb, Wrbf, Wi, bi, Wj, bj, riW1, rib1, riW2,
           rib2, u, Wm, bm, raW1, rab1, raW2, rab2, roW1, rob1, roW2, rob2,
           Wout):
    f32 = jnp.float32
    pad_ids = (jnp.arange(E_PAD - E, dtype=jnp.int32) % N)
    ii = jnp.concatenate([idx_i.astype(jnp.int32), pad_ids])
    jj = jnp.concatenate([idx_j.astype(jnp.int32), pad_ids])
    zp = jnp.pad(Z.astype(jnp.int32), (0, N_PAD - N))
    rx = jnp.pad(R[:, 0], (0, N_PAD - N))
    ry = jnp.pad(R[:, 1], (0, N_PAD - N))
    rz = jnp.pad(R[:, 2], (0, N_PAD - N))

    d2, x0 = _sc_prep(rx, ry, rz, zp, ii, jj, emb)
    d2r = d2.reshape(EPB, 128)
    iir = ii.reshape(NW * NCHUNK, CHUNK)
    jjr = jj.reshape(NW * (NCHUNK // 2), 128)

    x = x0
    eq = jnp.zeros((N_PAD, 2), f32)
    nh = f32(0.0)
    last = None
    for b in range(B):
        xi, hj = _tc_d1(x, Wi[b], bi[b].reshape(1, F), Wj[b], bj[b].reshape(1, F))
        g = _tc_g(d2r, Wrbf[b])
        xjp = _sc_agg(g, hj, iir, jjr)
        wts = (riW1[b, 0], riW1[b, 1], riW2[b, 0], riW2[b, 1],
               rib1[b, 0].reshape(1, F), rib1[b, 1].reshape(1, F),
               rib2[b, 0].reshape(1, F), rib2[b, 1].reshape(1, F),
               u[b].reshape(1, F), Wm[b], bm[b].reshape(1, F),
               raW1[b, 0], raW1[b, 1], raW2[b, 0], raW2[b, 1],
               rab1[b, 0].reshape(1, F), rab1[b, 1].reshape(1, F),
               rab2[b, 0].reshape(1, F), rab2[b, 1].reshape(1, F),
               roW1[b, 0], roW2[b, 0],
               rob1[b, 0].reshape(1, F), rob2[b, 0].reshape(1, F),
               Wout[b])
        res = _tc_d2(xi, xjp, x, eq, wts, last=last)
        if b == 0:
            x, eq, out2 = res
        else:
            x, eq, out2, nhp = res
            nh = nh + nhp[0, 0]
        last = out2
    return eq[:N, 0], eq[:N, 1], nh


# speedup vs baseline: 5.2201x; 1.0255x over previous
import dataclasses
import functools
import math

import jax
import jax.numpy as jnp
from jax import lax
from jax.experimental import pallas as pl
from jax.experimental.pallas import tpu as pltpu
from jax.experimental.pallas import tpu_sc as plsc

N = 10000
E = 160000
F = 128
K = 64
B = 3
RI = 2
RA = 2
RO = 1
CUT = 10.0

NC = 2
NS = 16
NW = NC * NS
E_PAD = 163840
EPW = E_PAD // NW
CHUNK = 64
NCHUNK = EPW // CHUNK
EPB = E_PAD // 128
N_PAD = 10240
RPW = N_PAD // NW
RPS = N_PAD // NS
TN = 1024
TE = 1024

_LOG2 = math.log(2.0)
_C0 = math.exp(-CUT)
_CD = (1.0 - math.exp(-CUT)) / (K - 1)
_WIDTH = (0.5 * K / (1.0 - math.exp(-CUT))) ** 2

_MESH = plsc.VectorSubcoreMesh(core_axis_name="c", subcore_axis_name="s")

_SC_CP = pltpu.CompilerParams()
if "needs_layout_passes" in pltpu.CompilerParams.__dataclass_fields__:
    _SC_CP = dataclasses.replace(_SC_CP, needs_layout_passes=False)


def _ssp(v):
    return jnp.maximum(v, 0.0) + jnp.log(1.0 + jnp.exp(-jnp.abs(v))) - _LOG2


def _sc_prep_body(rx_hbm, ry_hbm, rz_hbm, z_hbm, ii_hbm, jj_hbm, emb_hbm,
                  d2_hbm, x0_hbm,
                  rx_v, ry_v, rz_v, ii_v, jj_v, d2_v, z_v, xrows_v, sem):
    c = lax.axis_index("c")
    s = lax.axis_index("s")
    wid = s * NC + c
    pltpu.sync_copy(rx_hbm, rx_v)
    pltpu.sync_copy(ry_hbm, ry_v)
    pltpu.sync_copy(rz_hbm, rz_v)
    ebase = wid * EPW
    pltpu.sync_copy(ii_hbm.at[pl.ds(ebase, EPW)], ii_v)
    pltpu.sync_copy(jj_hbm.at[pl.ds(ebase, EPW)], jj_v)

    @pl.loop(0, EPW // 16)
    def _(ci):
        o = ci * 16
        iv = ii_v[pl.ds(o, 16)]
        jv = jj_v[pl.ds(o, 16)]
        dx = plsc.load_gather(rx_v, [iv]) - plsc.load_gather(rx_v, [jv])
        dy = plsc.load_gather(ry_v, [iv]) - plsc.load_gather(ry_v, [jv])
        dz = plsc.load_gather(rz_v, [iv]) - plsc.load_gather(rz_v, [jv])
        d2_v[pl.ds(o, 16)] = dx * dx + dy * dy + dz * dz

    pltpu.sync_copy(d2_v, d2_hbm.at[pl.ds(ebase, EPW)])
    nbase = wid * RPW
    pltpu.sync_copy(z_hbm.at[pl.ds(nbase, RPW)], z_v)
    pltpu.async_copy(emb_hbm.at[z_v], xrows_v, sem).wait()
    pltpu.sync_copy(xrows_v, x0_hbm.at[pl.ds(nbase, RPW)])


def _sc_prep(rx, ry, rz, zp, ii, jj, emb):
    k = pl.kernel(
        _sc_prep_body,
        out_type=[jax.ShapeDtypeStruct((E_PAD,), jnp.float32),
                  jax.ShapeDtypeStruct((N_PAD, F), jnp.float32)],
        mesh=_MESH,
        scratch_types=[pltpu.VMEM((N_PAD,), jnp.float32),
                       pltpu.VMEM((N_PAD,), jnp.float32),
                       pltpu.VMEM((N_PAD,), jnp.float32),
                       pltpu.VMEM((EPW,), jnp.int32),
                       pltpu.VMEM((EPW,), jnp.int32),
                       pltpu.VMEM((EPW,), jnp.float32),
                       pltpu.VMEM((RPW,), jnp.int32),
                       pltpu.VMEM((RPW, F), jnp.float32),
                       pltpu.SemaphoreType.DMA],
        compiler_params=_SC_CP,
    )
    return k(rx, ry, rz, zp, ii, jj, emb)


def _sc_agg_body(g_hbm, hj_hbm, iir_hbm, jjr_hbm, xjp_hbm,
                 acc_sh, jj_v, iib0, iib1, iib2, iib3, gb0, gb1, hb0, hb1, pb,
                 gsem0, gsem1, hsem0, hsem1, isem0, isem1, isem2, isem3, ssem):
    c = lax.axis_index("c")
    s = lax.axis_index("s")
    wid = s * NC + c
    gb = (gb0, gb1)
    hb = (hb0, hb1)
    iib = (iib0, iib1, iib2, iib3)
    gsem = (gsem0, gsem1)
    hsem = (hsem0, hsem1)
    isem = (isem0, isem1, isem2, isem3)

    @pl.loop(0, CHUNK)
    def _(r):
        for q in range(8):
            pb[r, pl.ds(q * 16, 16)] = jnp.zeros((16,), jnp.float32)

    @pl.loop(0, RPS // CHUNK)
    def _(kk):
        pltpu.sync_copy(pb, acc_sh.at[pl.ds(s * RPS + kk * CHUNK, CHUNK)])

    pltpu.sync_copy(jjr_hbm.at[pl.ds(wid * (NCHUNK // 2), NCHUNK // 2)], jj_v)
    for b in range(2):
        base = wid * EPW + b * CHUNK
        pltpu.async_copy(g_hbm.at[pl.ds(base, CHUNK)], gb[b], gsem[b])
        pltpu.async_copy(hj_hbm.at[jj_v.at[0, pl.ds(b * CHUNK, CHUNK)]],
                         hb[b], hsem[b])
        pltpu.async_copy(iir_hbm.at[pl.ds(wid * NCHUNK + b, 1)], iib[b],
                         isem[b])
    plsc.subcore_barrier()

    @pl.loop(0, NCHUNK // 4)
    def _(rr):
        for b in range(4):
            db = b % 2
            k = rr * 4 + b
            pltpu.make_async_copy(g_hbm.at[pl.ds(0, CHUNK)], gb[db],
                                  gsem[db]).wait()
            pltpu.make_async_copy(g_hbm.at[pl.ds(0, CHUNK)], hb[db],
                                  hsem[db]).wait()
            pltpu.make_async_copy(iir_hbm.at[pl.ds(0, 1)], iib[b],
                                  isem[b]).wait()

            @pl.when(k >= 1)
            def _():
                pltpu.make_async_copy(pb, acc_sh.at[pl.ds(0, CHUNK)],
                                      ssem).wait()

            @pl.loop(0, CHUNK, step=2)
            def _(r):
                for dr in range(2):
                    for q in range(8):
                        sl = pl.ds(q * 16, 16)
                        pb[r + dr, sl] = gb[db][r + dr, sl] * hb[db][r + dr, sl]

            @pl.when(k + 2 < NCHUNK)
            def _():
                base2 = wid * EPW + (k + 2) * CHUNK
                pltpu.async_copy(g_hbm.at[pl.ds(base2, CHUNK)], gb[db],
                                 gsem[db])
                pltpu.async_copy(
                    hj_hbm.at[jj_v.at[(k + 2) // 2,
                                      pl.ds(((k + 2) % 2) * CHUNK, CHUNK)]],
                    hb[db], hsem[db])
                pltpu.async_copy(iir_hbm.at[pl.ds(wid * NCHUNK + k + 2, 1)],
                                 iib[(b + 2) % 4], isem[(b + 2) % 4])

            pltpu.async_copy(pb, acc_sh.at[iib[b].at[0]], ssem, add=True)

    pltpu.make_async_copy(pb, acc_sh.at[pl.ds(0, CHUNK)], ssem).wait()
    plsc.subcore_barrier()
    pltpu.sync_copy(acc_sh.at[pl.ds(s * RPS, RPS)],
                    xjp_hbm.at[c, pl.ds(s * RPS, RPS)])


def _sc_agg(g, hj, iir, jjr):
    k = pl.kernel(
        _sc_agg_body,
        out_type=jax.ShapeDtypeStruct((NC, N_PAD, F), jnp.float32),
        mesh=_MESH,
        scratch_types=[pltpu.VMEM_SHARED((N_PAD, F), jnp.float32),
                       pltpu.VMEM((NCHUNK // 2, 128), jnp.int32),
                       pltpu.VMEM((1, CHUNK), jnp.int32),
                       pltpu.VMEM((1, CHUNK), jnp.int32),
                       pltpu.VMEM((1, CHUNK), jnp.int32),
                       pltpu.VMEM((1, CHUNK), jnp.int32),
                       pltpu.VMEM((CHUNK, F), jnp.float32),
                       pltpu.VMEM((CHUNK, F), jnp.float32),
                       pltpu.VMEM((CHUNK, F), jnp.float32),
                       pltpu.VMEM((CHUNK, F), jnp.float32),
                       pltpu.VMEM((CHUNK, F), jnp.float32),
                       pltpu.SemaphoreType.DMA,
                       pltpu.SemaphoreType.DMA,
                       pltpu.SemaphoreType.DMA,
                       pltpu.SemaphoreType.DMA,
                       pltpu.SemaphoreType.DMA,
                       pltpu.SemaphoreType.DMA,
                       pltpu.SemaphoreType.DMA,
                       pltpu.SemaphoreType.DMA,
                       pltpu.SemaphoreType.DMA],
        compiler_params=_SC_CP,
    )
    return k(g, hj, iir, jjr)


def _tc_g_body(d2_ref, w_ref, t_ref):
    pid = pl.program_id(0)
    d2 = d2_ref[...]
    d = jnp.sqrt(d2 + 1e-12)
    r = d / CUT
    r2 = r * r
    r3 = r2 * r
    poly = 1.0 - 6.0 * r3 * r2 + 15.0 * r2 * r2 - 10.0 * r3
    eid = (pid * TE
           + lax.broadcasted_iota(jnp.int32, (8, 128), 0) * 128
           + lax.broadcasted_iota(jnp.int32, (8, 128), 1))
    cut = jnp.where((d < CUT) & (eid < E), poly, 0.0)
    en = jnp.exp(-d)
    ck = _C0 + _CD * lax.broadcasted_iota(jnp.int32, (K, 1), 0).astype(jnp.float32)
    w = w_ref[...]
    for rr in range(8):
        en_r = jnp.broadcast_to(en[rr:rr + 1, :], (K, 128))
        cut_r = jnp.broadcast_to(cut[rr:rr + 1, :], (K, 128))
        diff = en_r - ck
        rbf_t = cut_r * jnp.exp(-_WIDTH * diff * diff)
        g = lax.dot_general(rbf_t, w, (((0,), (0,)), ((), ())),
                            preferred_element_type=jnp.float32)
        t_ref[pl.ds(rr * 128, 128), :] = g


def _tc_g(d2r, wrbf):
    return pl.pallas_call(
        _tc_g_body,
        grid=(E_PAD // TE,),
        in_specs=[pl.BlockSpec((8, 128), lambda i: (i, 0)),
                  pl.BlockSpec((K, F), lambda i: (0, 0))],
        out_specs=pl.BlockSpec((TE, F), lambda i: (i, 0)),
        out_shape=jax.ShapeDtypeStruct((E_PAD, F), jnp.float32),
    )(d2r, wrbf)


def _tc_d1_body(x_ref, wi_ref, bi_ref, wj_ref, bj_ref, xi_ref, hj_ref):
    xa = _ssp(x_ref[...])
    xi_ref[...] = _ssp(jnp.dot(xa, wi_ref[...],
                               preferred_element_type=jnp.float32) + bi_ref[...])
    hj_ref[...] = _ssp(jnp.dot(xa, wj_ref[...],
                               preferred_element_type=jnp.float32) + bj_ref[...])


def _tc_d1(x, wi, bi, wj, bj):
    return pl.pallas_call(
        _tc_d1_body,
        grid=(N_PAD // TN,),
        in_specs=[pl.BlockSpec((TN, F), lambda i: (i, 0)),
                  pl.BlockSpec((F, F), lambda i: (0, 0)),
                  pl.BlockSpec((1, F), lambda i: (0, 0)),
                  pl.BlockSpec((F, F), lambda i: (0, 0)),
                  pl.BlockSpec((1, F), lambda i: (0, 0))],
        out_specs=[pl.BlockSpec((TN, F), lambda i: (i, 0)),
                   pl.BlockSpec((TN, F), lambda i: (i, 0))],
        out_shape=[jax.ShapeDtypeStruct((N_PAD, F), jnp.float32),
                   jax.ShapeDtypeStruct((N_PAD, F), jnp.float32)],
    )(x, wi, bi, wj, bj)


def _mm(a, w_ref):
    return jnp.dot(a, w_ref[...], preferred_element_type=jnp.float32)


def _tc_d2_body(has_prev, xi_ref, xj0_ref, xj1_ref, x_ref, eq_ref,
                ri10_ref, ri11_ref, ri20_ref, ri21_ref,
                rib10_ref, rib11_ref, rib20_ref, rib21_ref,
                u_ref, wm_ref, bm_ref,
                ra10_ref, ra11_ref, ra20_ref, ra21_ref,
                rab10_ref, rab11_ref, rab20_ref, rab21_ref,
                ro1_ref, ro2_ref, rob1_ref, rob2_ref,
                wout_ref, *rest):
    if has_prev:
        last_ref = rest[0]
        xo_ref, eqo_ref, o2_ref, nh_ref = rest[1:]
    else:
        xo_ref, eqo_ref, o2_ref = rest

    m = xi_ref[...] + xj0_ref[...] + xj1_ref[...]
    for w1, b1, w2, b2 in ((ri10_ref, rib10_ref, ri20_ref, rib20_ref),
                           (ri11_ref, rib11_ref, ri21_ref, rib21_ref)):
        ma = _ssp(m)
        m = m + _mm(_ssp(_mm(ma, w1) + b1[...]), w2) + b2[...]
    m = _ssp(m)
    x = u_ref[...] * x_ref[...] + _mm(m, wm_ref) + bm_ref[...]
    for w1, b1, w2, b2 in ((ra10_ref, rab10_ref, ra20_ref, rab20_ref),
                           (ra11_ref, rab11_ref, ra21_ref, rab21_ref)):
        xa2 = _ssp(x)
        x = x + _mm(_ssp(_mm(xa2, w1) + b1[...]), w2) + b2[...]
    xo_ref[...] = x
    o = x + _mm(_ssp(_mm(_ssp(x), ro1_ref) + rob1_ref[...]), ro2_ref) + rob2_ref[...]
    out = _mm(_ssp(o), wout_ref)
    eqo_ref[...] = eq_ref[...] + out
    o2 = out * out
    o2_ref[...] = o2
    if has_prev:
        pid = pl.program_id(0)
        rows = pid * TN + lax.broadcasted_iota(jnp.int32, (TN, 2), 0)
        ratio = jnp.where(rows < N, o2 / (o2 + last_ref[...] + 1e-7), 0.0)
        part = (jnp.sum(ratio) / (N * 2.0)).reshape(1, 1)

        @pl.when(pid == 0)
        def _():
            nh_ref[...] = part

        @pl.when(pid > 0)
        def _():
            nh_ref[...] += part


def _tc_d2(xi, xjp, x, eq, wts, last=None):
    has_prev = last is not None
    full = lambda shape: pl.BlockSpec(shape, lambda i: tuple(0 for _ in shape))
    in_specs = [pl.BlockSpec((TN, F), lambda i: (i, 0)),
                pl.BlockSpec((TN, F), lambda i: (i, 0)),
                pl.BlockSpec((TN, F), lambda i: (i, 0)),
                pl.BlockSpec((TN, F), lambda i: (i, 0)),
                pl.BlockSpec((TN, 2), lambda i: (i, 0))]
    wspecs = []
    for warr in wts:
        wspecs.append(full(tuple(warr.shape)))
    in_specs += wspecs
    out_specs = [pl.BlockSpec((TN, F), lambda i: (i, 0)),
                 pl.BlockSpec((TN, 2), lambda i: (i, 0)),
                 pl.BlockSpec((TN, 2), lambda i: (i, 0))]
    out_shape = [jax.ShapeDtypeStruct((N_PAD, F), jnp.float32),
                 jax.ShapeDtypeStruct((N_PAD, 2), jnp.float32),
                 jax.ShapeDtypeStruct((N_PAD, 2), jnp.float32)]
    args = [xi, xjp[0], xjp[1], x, eq] + list(wts)
    if has_prev:
        in_specs.append(pl.BlockSpec((TN, 2), lambda i: (i, 0)))
        out_specs.append(pl.BlockSpec((1, 1), lambda i: (0, 0)))
        out_shape.append(jax.ShapeDtypeStruct((1, 1), jnp.float32))
        args.append(last)
    return pl.pallas_call(
        functools.partial(_tc_d2_body, has_prev),
        grid=(N_PAD // TN,),
        in_specs=in_specs,
        out_specs=out_specs,
        out_shape=out_shape,
    )(*args)


def kernel(R, Z, idx_i, idx_j, emb, Wrbf, Wi, bi, Wj, bj, riW1, rib1, riW2,
           rib2, u, Wm, bm, raW1, rab1, raW2, rab2, roW1, rob1, roW2, rob2,
           Wout):
    f32 = jnp.float32
    pad_ids = (jnp.arange(E_PAD - E, dtype=jnp.int32) % N)
    ii = jnp.concatenate([idx_i.astype(jnp.int32), pad_ids])
    jj = jnp.concatenate([idx_j.astype(jnp.int32), pad_ids])
    zp = jnp.pad(Z.astype(jnp.int32), (0, N_PAD - N))
    rx = jnp.pad(R[:, 0], (0, N_PAD - N))
    ry = jnp.pad(R[:, 1], (0, N_PAD - N))
    rz = jnp.pad(R[:, 2], (0, N_PAD - N))

    d2, x0 = _sc_prep(rx, ry, rz, zp, ii, jj, emb)
    d2r = d2.reshape(EPB, 128)
    iir = ii.reshape(NW * NCHUNK, CHUNK)
    jjr = jj.reshape(NW * (NCHUNK // 2), 128)

    gs = [_tc_g(d2r, Wrbf[b]) for b in range(B)]

    x = x0
    eq = jnp.zeros((N_PAD, 2), f32)
    nh = f32(0.0)
    last = None
    for b in range(B):
        xi, hj = _tc_d1(x, Wi[b], bi[b].reshape(1, F), Wj[b], bj[b].reshape(1, F))
        xjp = _sc_agg(gs[b], hj, iir, jjr)
        wts = (riW1[b, 0], riW1[b, 1], riW2[b, 0], riW2[b, 1],
               rib1[b, 0].reshape(1, F), rib1[b, 1].reshape(1, F),
               rib2[b, 0].reshape(1, F), rib2[b, 1].reshape(1, F),
               u[b].reshape(1, F), Wm[b], bm[b].reshape(1, F),
               raW1[b, 0], raW1[b, 1], raW2[b, 0], raW2[b, 1],
               rab1[b, 0].reshape(1, F), rab1[b, 1].reshape(1, F),
               rab2[b, 0].reshape(1, F), rab2[b, 1].reshape(1, F),
               roW1[b, 0], roW2[b, 0],
               rob1[b, 0].reshape(1, F), rob2[b, 0].reshape(1, F),
               Wout[b])
        res = _tc_d2(xi, xjp, x, eq, wts, last=last)
        if b == 0:
            x, eq, out2 = res
        else:
            x, eq, out2, nhp = res
            nh = nh + nhp[0, 0]
        last = out2
    return eq[:N, 0], eq[:N, 1], nh


# speedup vs baseline: 5.4277x; 1.0398x over previous
import dataclasses
import functools
import math

import jax
import jax.numpy as jnp
import numpy as np
from jax import lax
from jax.experimental import pallas as pl
from jax.experimental.pallas import tpu as pltpu
from jax.experimental.pallas import tpu_sc as plsc

N = 10000
E = 160000
F = 128
K = 64
B = 3
RI = 2
RA = 2
RO = 1
CUT = 10.0

NC = 2
NS = 16
NW = NC * NS
E_PAD = 163840
EPW = E_PAD // NW
CHUNK = 64
NCHUNK = EPW // CHUNK
EPB = E_PAD // 128
N_PAD = 10240
RPW = N_PAD // NW
RPS = N_PAD // NS
TN = 1024
TE = 1024

_LOG2 = math.log(2.0)
_C0 = math.exp(-CUT)
_CD = (1.0 - math.exp(-CUT)) / (K - 1)
_WIDTH = (0.5 * K / (1.0 - math.exp(-CUT))) ** 2

_MESH = plsc.VectorSubcoreMesh(core_axis_name="c", subcore_axis_name="s")

_SC_CP = pltpu.CompilerParams()
if "needs_layout_passes" in pltpu.CompilerParams.__dataclass_fields__:
    _SC_CP = dataclasses.replace(_SC_CP, needs_layout_passes=False)


def _ssp(v):
    return jnp.maximum(v, 0.0) + jnp.log(1.0 + jnp.exp(-jnp.abs(v))) - _LOG2


def _sc_prep_body(rx_hbm, ry_hbm, rz_hbm, z_hbm, ii_hbm, jj_hbm, emb_hbm,
                  d2_hbm, x0_hbm,
                  rx_v, ry_v, rz_v, ii_v, jj_v, d2_v, z_v, xrows_v, sem):
    c = lax.axis_index("c")
    s = lax.axis_index("s")
    wid = s * NC + c
    pltpu.sync_copy(rx_hbm, rx_v)
    pltpu.sync_copy(ry_hbm, ry_v)
    pltpu.sync_copy(rz_hbm, rz_v)
    ebase = wid * EPW
    pltpu.sync_copy(ii_hbm.at[pl.ds(ebase, EPW)], ii_v)
    pltpu.sync_copy(jj_hbm.at[pl.ds(ebase, EPW)], jj_v)

    @pl.loop(0, EPW // 16)
    def _(ci):
        o = ci * 16
        iv = ii_v[pl.ds(o, 16)]
        jv = jj_v[pl.ds(o, 16)]
        dx = plsc.load_gather(rx_v, [iv]) - plsc.load_gather(rx_v, [jv])
        dy = plsc.load_gather(ry_v, [iv]) - plsc.load_gather(ry_v, [jv])
        dz = plsc.load_gather(rz_v, [iv]) - plsc.load_gather(rz_v, [jv])
        d2_v[pl.ds(o, 16)] = dx * dx + dy * dy + dz * dz

    pltpu.sync_copy(d2_v, d2_hbm.at[pl.ds(ebase, EPW)])
    nbase = wid * RPW
    pltpu.sync_copy(z_hbm.at[pl.ds(nbase, RPW)], z_v)
    pltpu.async_copy(emb_hbm.at[z_v], xrows_v, sem).wait()
    pltpu.sync_copy(xrows_v, x0_hbm.at[pl.ds(nbase, RPW)])


def _sc_prep(rx, ry, rz, zp, ii, jj, emb):
    k = pl.kernel(
        _sc_prep_body,
        out_type=[jax.ShapeDtypeStruct((E_PAD,), jnp.float32),
                  jax.ShapeDtypeStruct((N_PAD, F), jnp.float32)],
        mesh=_MESH,
        scratch_types=[pltpu.VMEM((N_PAD,), jnp.float32),
                       pltpu.VMEM((N_PAD,), jnp.float32),
                       pltpu.VMEM((N_PAD,), jnp.float32),
                       pltpu.VMEM((EPW,), jnp.int32),
                       pltpu.VMEM((EPW,), jnp.int32),
                       pltpu.VMEM((EPW,), jnp.float32),
                       pltpu.VMEM((RPW,), jnp.int32),
                       pltpu.VMEM((RPW, F), jnp.float32),
                       pltpu.SemaphoreType.DMA],
        compiler_params=_SC_CP,
    )
    return k(rx, ry, rz, zp, ii, jj, emb)


def _sc_agg_body(g_hbm, hj_hbm, iir_hbm, jjr_hbm, xjp_hbm,
                 acc_sh, jj_v, iib0, iib1, iib2, iib3, gb0, gb1, hb0, hb1, pb,
                 gsem0, gsem1, hsem0, hsem1, isem0, isem1, isem2, isem3, ssem):
    c = lax.axis_index("c")
    s = lax.axis_index("s")
    wid = s * NC + c
    gb = (gb0, gb1)
    hb = (hb0, hb1)
    iib = (iib0, iib1, iib2, iib3)
    gsem = (gsem0, gsem1)
    hsem = (hsem0, hsem1)
    isem = (isem0, isem1, isem2, isem3)

    @pl.loop(0, CHUNK)
    def _(r):
        for q in range(8):
            pb[r, pl.ds(q * 16, 16)] = jnp.zeros((16,), jnp.float32)

    @pl.loop(0, RPS // CHUNK)
    def _(kk):
        pltpu.sync_copy(pb, acc_sh.at[pl.ds(s * RPS + kk * CHUNK, CHUNK)])

    pltpu.sync_copy(jjr_hbm.at[pl.ds(wid * (NCHUNK // 2), NCHUNK // 2)], jj_v)
    for b in range(2):
        base = wid * EPW + b * CHUNK
        pltpu.async_copy(g_hbm.at[pl.ds(base, CHUNK)], gb[b], gsem[b])
        pltpu.async_copy(hj_hbm.at[jj_v.at[0, pl.ds(b * CHUNK, CHUNK)]],
                         hb[b], hsem[b])
        pltpu.async_copy(iir_hbm.at[pl.ds(wid * NCHUNK + b, 1)], iib[b],
                         isem[b])
    plsc.subcore_barrier()

    @pl.loop(0, NCHUNK // 4)
    def _(rr):
        for b in range(4):
            db = b % 2
            k = rr * 4 + b
            pltpu.make_async_copy(g_hbm.at[pl.ds(0, CHUNK)], gb[db],
                                  gsem[db]).wait()
            pltpu.make_async_copy(hj_hbm.at[pl.ds(0, CHUNK)], hb[db],
                                  hsem[db]).wait()
            pltpu.make_async_copy(iir_hbm.at[pl.ds(0, 1)], iib[b],
                                  isem[b]).wait()

            @pl.when(k >= 1)
            def _():
                pltpu.make_async_copy(pb, acc_sh.at[pl.ds(0, CHUNK)],
                                      ssem).wait()

            @pl.loop(0, CHUNK)
            def _(r):
                for q in range(8):
                    sl = pl.ds(q * 16, 16)
                    pb[r, sl] = gb[db][r, sl] * hb[db][r, sl]

            @pl.when(k + 2 < NCHUNK)
            def _():
                base2 = wid * EPW + (k + 2) * CHUNK
                pltpu.async_copy(g_hbm.at[pl.ds(base2, CHUNK)], gb[db],
                                 gsem[db])
                pltpu.async_copy(
                    hj_hbm.at[jj_v.at[rr * 2 + 1 + (b // 2),
                                      pl.ds((b % 2) * CHUNK, CHUNK)]],
                    hb[db], hsem[db])
                pltpu.async_copy(iir_hbm.at[pl.ds(wid * NCHUNK + k + 2, 1)],
                                 iib[(b + 2) % 4], isem[(b + 2) % 4])

            pltpu.async_copy(pb, acc_sh.at[iib[b].at[0]], ssem, add=True)

    pltpu.make_async_copy(pb, acc_sh.at[pl.ds(0, CHUNK)], ssem).wait()
    plsc.subcore_barrier()
    pltpu.sync_copy(acc_sh.at[pl.ds(s * RPS, RPS)],
                    xjp_hbm.at[c, pl.ds(s * RPS, RPS)])


def _sc_agg(g, hj, iir, jjr):
    k = pl.kernel(
        _sc_agg_body,
        out_type=jax.ShapeDtypeStruct((NC, N_PAD, F), jnp.float32),
        mesh=_MESH,
        scratch_types=[pltpu.VMEM_SHARED((N_PAD, F), jnp.float32),
                       pltpu.VMEM((NCHUNK // 2, 128), jnp.int32),
                       pltpu.VMEM((1, CHUNK), jnp.int32),
                       pltpu.VMEM((1, CHUNK), jnp.int32),
                       pltpu.VMEM((1, CHUNK), jnp.int32),
                       pltpu.VMEM((1, CHUNK), jnp.int32),
                       pltpu.VMEM((CHUNK, F), jnp.float32),
                       pltpu.VMEM((CHUNK, F), jnp.float32),
                       pltpu.VMEM((CHUNK, F), jnp.float32),
                       pltpu.VMEM((CHUNK, F), jnp.float32),
                       pltpu.VMEM((CHUNK, F), jnp.float32),
                       pltpu.SemaphoreType.DMA,
                       pltpu.SemaphoreType.DMA,
                       pltpu.SemaphoreType.DMA,
                       pltpu.SemaphoreType.DMA,
                       pltpu.SemaphoreType.DMA,
                       pltpu.SemaphoreType.DMA,
                       pltpu.SemaphoreType.DMA,
                       pltpu.SemaphoreType.DMA,
                       pltpu.SemaphoreType.DMA],
        compiler_params=_SC_CP,
    )
    return k(g, hj, iir, jjr)


def _tc_g_body(d2_ref, w0_ref, w1_ref, w2_ref, t0_ref, t1_ref, t2_ref):
    pid = pl.program_id(0)
    d2 = d2_ref[...]
    d = jnp.sqrt(d2 + 1e-12)
    r = d / CUT
    r2 = r * r
    r3 = r2 * r
    poly = 1.0 - 6.0 * r3 * r2 + 15.0 * r2 * r2 - 10.0 * r3
    eid = (pid * TE
           + lax.broadcasted_iota(jnp.int32, (8, 128), 0) * 128
           + lax.broadcasted_iota(jnp.int32, (8, 128), 1))
    cut = jnp.where((d < CUT) & (eid < E), poly, 0.0)
    en = jnp.exp(-d)
    ck = _C0 + _CD * lax.broadcasted_iota(jnp.int32, (K, 1), 0).astype(jnp.float32)
    for rr in range(8):
        en_r = jnp.broadcast_to(en[rr:rr + 1, :], (K, 128))
        cut_r = jnp.broadcast_to(cut[rr:rr + 1, :], (K, 128))
        diff = en_r - ck
        rbf_t = cut_r * jnp.exp(-_WIDTH * diff * diff)
        sl = pl.ds(rr * 128, 128)
        for w_ref, t_ref in ((w0_ref, t0_ref), (w1_ref, t1_ref),
                             (w2_ref, t2_ref)):
            t_ref[sl, :] = lax.dot_general(
                rbf_t, w_ref[...], (((0,), (0,)), ((), ())),
                preferred_element_type=jnp.float32)


def _tc_g3(d2r, wrbf):
    out = jax.ShapeDtypeStruct((E_PAD, F), jnp.float32)
    wspec = pl.BlockSpec((K, F), lambda i: (0, 0))
    espec = pl.BlockSpec((TE, F), lambda i: (i, 0))
    return pl.pallas_call(
        _tc_g_body,
        grid=(E_PAD // TE,),
        in_specs=[pl.BlockSpec((8, 128), lambda i: (i, 0)),
                  wspec, wspec, wspec],
        out_specs=[espec, espec, espec],
        out_shape=[out, out, out],
    )(d2r, wrbf[0], wrbf[1], wrbf[2])


def _tc_d1_body(x_ref, wi_ref, bi_ref, wj_ref, bj_ref, xi_ref, hj_ref):
    xa = _ssp(x_ref[...])
    xi_ref[...] = _ssp(jnp.dot(xa, wi_ref[...],
                               preferred_element_type=jnp.float32) + bi_ref[...])
    hj_ref[...] = _ssp(jnp.dot(xa, wj_ref[...],
                               preferred_element_type=jnp.float32) + bj_ref[...])


def _tc_d1(x, wi, bi, wj, bj):
    return pl.pallas_call(
        _tc_d1_body,
        grid=(N_PAD // TN,),
        in_specs=[pl.BlockSpec((TN, F), lambda i: (i, 0)),
                  pl.BlockSpec((F, F), lambda i: (0, 0)),
                  pl.BlockSpec((1, F), lambda i: (0, 0)),
                  pl.BlockSpec((F, F), lambda i: (0, 0)),
                  pl.BlockSpec((1, F), lambda i: (0, 0))],
        out_specs=[pl.BlockSpec((TN, F), lambda i: (i, 0)),
                   pl.BlockSpec((TN, F), lambda i: (i, 0))],
        out_shape=[jax.ShapeDtypeStruct((N_PAD, F), jnp.float32),
                   jax.ShapeDtypeStruct((N_PAD, F), jnp.float32)],
    )(x, wi, bi, wj, bj)


def _mm(a, w_ref):
    return jnp.dot(a, w_ref[...], preferred_element_type=jnp.float32)


def _tc_d2_body(has_prev, has_next, xi_ref, xj0_ref, xj1_ref, x_ref, eq_ref,
                ri10_ref, ri11_ref, ri20_ref, ri21_ref,
                rib10_ref, rib11_ref, rib20_ref, rib21_ref,
                u_ref, wm_ref, bm_ref,
                ra10_ref, ra11_ref, ra20_ref, ra21_ref,
                rab10_ref, rab11_ref, rab20_ref, rab21_ref,
                ro1_ref, ro2_ref, rob1_ref, rob2_ref,
                wout_ref, *rest):
    rest = list(rest)
    if has_next:
        wi_ref, bi_ref, wj_ref, bj_ref = rest[:4]
        rest = rest[4:]
    if has_prev:
        last_ref = rest.pop(0)
    xo_ref, eqo_ref, o2_ref = rest[:3]
    rest = rest[3:]
    if has_prev:
        nh_ref = rest.pop(0)
    if has_next:
        xi2_ref, hj2_ref = rest[:2]

    m = xi_ref[...] + xj0_ref[...] + xj1_ref[...]
    for w1, b1, w2, b2 in ((ri10_ref, rib10_ref, ri20_ref, rib20_ref),
                           (ri11_ref, rib11_ref, ri21_ref, rib21_ref)):
        ma = _ssp(m)
        m = m + _mm(_ssp(_mm(ma, w1) + b1[...]), w2) + b2[...]
    m = _ssp(m)
    x = u_ref[...] * x_ref[...] + _mm(m, wm_ref) + bm_ref[...]
    for w1, b1, w2, b2 in ((ra10_ref, rab10_ref, ra20_ref, rab20_ref),
                           (ra11_ref, rab11_ref, ra21_ref, rab21_ref)):
        xa2 = _ssp(x)
        x = x + _mm(_ssp(_mm(xa2, w1) + b1[...]), w2) + b2[...]
    xo_ref[...] = x
    if has_next:
        xa = _ssp(x)
        xi2_ref[...] = _ssp(_mm(xa, wi_ref) + bi_ref[...])
        hj2_ref[...] = _ssp(_mm(xa, wj_ref) + bj_ref[...])
    o = x + _mm(_ssp(_mm(_ssp(x), ro1_ref) + rob1_ref[...]), ro2_ref) + rob2_ref[...]
    out = _mm(_ssp(o), wout_ref)
    eqo_ref[...] = eq_ref[...] + out
    o2 = out * out
    o2_ref[...] = o2
    if has_prev:
        pid = pl.program_id(0)
        rows = pid * TN + lax.broadcasted_iota(jnp.int32, (TN, 2), 0)
        ratio = jnp.where(rows < N, o2 / (o2 + last_ref[...] + 1e-7), 0.0)
        part = (jnp.sum(ratio) / (N * 2.0)).reshape(1, 1)

        @pl.when(pid == 0)
        def _():
            nh_ref[...] = part

        @pl.when(pid > 0)
        def _():
            nh_ref[...] += part


def _tc_d2(xi, xjp, x, eq, wts, last=None, nxt=None):
    has_prev = last is not None
    has_next = nxt is not None
    full = lambda shape: pl.BlockSpec(shape, lambda i: tuple(0 for _ in shape))
    row = pl.BlockSpec((TN, F), lambda i: (i, 0))
    row2 = pl.BlockSpec((TN, 2), lambda i: (i, 0))
    in_specs = [row, row, row, row, row2]
    for warr in wts:
        in_specs.append(full(tuple(warr.shape)))
    args = [xi, xjp[0], xjp[1], x, eq] + list(wts)
    if has_next:
        for warr in nxt:
            in_specs.append(full(tuple(warr.shape)))
        args += list(nxt)
    if has_prev:
        in_specs.append(row2)
        args.append(last)
    out_specs = [row, row2, row2]
    out_shape = [jax.ShapeDtypeStruct((N_PAD, F), jnp.float32),
                 jax.ShapeDtypeStruct((N_PAD, 2), jnp.float32),
                 jax.ShapeDtypeStruct((N_PAD, 2), jnp.float32)]
    if has_prev:
        out_specs.append(pl.BlockSpec((1, 1), lambda i: (0, 0)))
        out_shape.append(jax.ShapeDtypeStruct((1, 1), jnp.float32))
    if has_next:
        out_specs += [row, row]
        out_shape += [jax.ShapeDtypeStruct((N_PAD, F), jnp.float32),
                      jax.ShapeDtypeStruct((N_PAD, F), jnp.float32)]
    return pl.pallas_call(
        functools.partial(_tc_d2_body, has_prev, has_next),
        grid=(N_PAD // TN,),
        in_specs=in_specs,
        out_specs=out_specs,
        out_shape=out_shape,
    )(*args)


def kernel(R, Z, idx_i, idx_j, emb, Wrbf, Wi, bi, Wj, bj, riW1, rib1, riW2,
           rib2, u, Wm, bm, raW1, rab1, raW2, rab2, roW1, rob1, roW2, rob2,
           Wout):
    f32 = jnp.float32
    pad_ids = (jnp.arange(E_PAD - E, dtype=jnp.int32) % N)
    ii = jnp.concatenate([idx_i.astype(jnp.int32), pad_ids])
    jj = jnp.concatenate([idx_j.astype(jnp.int32), pad_ids])
    zp = jnp.pad(Z.astype(jnp.int32), (0, N_PAD - N))
    rx = jnp.pad(R[:, 0], (0, N_PAD - N))
    ry = jnp.pad(R[:, 1], (0, N_PAD - N))
    rz = jnp.pad(R[:, 2], (0, N_PAD - N))

    d2, x0 = _sc_prep(rx, ry, rz, zp, ii, jj, emb)
    d2r = d2.reshape(EPB, 128)
    iir = ii.reshape(NW * NCHUNK, CHUNK)
    jjr = jj.reshape(NW * (NCHUNK // 2), 128)

    gs = _tc_g3(d2r, Wrbf)

    x = x0
    eq = jnp.zeros((N_PAD, 2), f32)
    nh = f32(0.0)
    last = None
    xi, hj = _tc_d1(x0, Wi[0], bi[0].reshape(1, F),
                    Wj[0], bj[0].reshape(1, F))
    for b in range(B):
        xjp = _sc_agg(gs[b], hj, iir, jjr)
        wts = (riW1[b, 0], riW1[b, 1], riW2[b, 0], riW2[b, 1],
               rib1[b, 0].reshape(1, F), rib1[b, 1].reshape(1, F),
               rib2[b, 0].reshape(1, F), rib2[b, 1].reshape(1, F),
               u[b].reshape(1, F), Wm[b], bm[b].reshape(1, F),
               raW1[b, 0], raW1[b, 1], raW2[b, 0], raW2[b, 1],
               rab1[b, 0].reshape(1, F), rab1[b, 1].reshape(1, F),
               rab2[b, 0].reshape(1, F), rab2[b, 1].reshape(1, F),
               roW1[b, 0], roW2[b, 0],
               rob1[b, 0].reshape(1, F), rob2[b, 0].reshape(1, F),
               Wout[b])
        nxt = None
        if b + 1 < B:
            nxt = (Wi[b + 1], bi[b + 1].reshape(1, F),
                   Wj[b + 1], bj[b + 1].reshape(1, F))
        res = _tc_d2(xi, xjp, x, eq, wts, last=last, nxt=nxt)
        if b == 0:
            x, eq, out2, xi, hj = res
        elif b == 1:
            x, eq, out2, nhp, xi, hj = res
            nh = nh + nhp[0, 0]
        else:
            x, eq, out2, nhp = res
            nh = nh + nhp[0, 0]
        last = out2
    return eq[:N, 0], eq[:N, 1], nh
